# baseline (speedup 1.0000x reference)
import jax
import jax.numpy as jnp
from jax import lax
from jax.experimental import pallas as pl
from jax.experimental.pallas import tpu as pltpu
from jax.experimental.pallas import tpu_sc as plsc

_R, _N = 128, 32768
_L = 16
_NCH = _N // _L
_NW = 32
_RPW = _R // _NW
_NB = 256
_NG = _NB // _L
_LEVELS = 2


def _splat_last(x):
    idx = jnp.full((_L,), _L - 1, jnp.int32)
    return x.at[idx].get(mode="promise_in_bounds")


def _sc_body(in_hbm, out_hbm, buf, hcnt, hsum):
    cid = lax.axis_index("c")
    sid = lax.axis_index("s")
    wid = sid * 2 + cid
    lanes = lax.iota(jnp.int32, _L)
    lanes_f = lanes.astype(jnp.float32)
    zeros = jnp.zeros((_L,), jnp.float32)
    ones = jnp.ones((_L,), jnp.float32)

    for r in range(_RPW):
        row = wid * _RPW + r
        pltpu.sync_copy(in_hbm.at[row], buf)

        @plsc.parallel_loop(0, _NCH, carry=jnp.full((_L,), -jnp.inf, jnp.float32), unroll=4)
        def _mx(i, acc):
            return jnp.maximum(acc, buf[pl.ds(i * _L, _L)])

        m = _splat_last(plsc.cummax(_mx))
        lo = m - 1.0
        hi = m
        cc_star = ones
        ss_star = zeros

        for lev in range(_LEVELS):
            inv_bw = jnp.float32(2.0 ** (8 + 6 * lev))
            bw = jnp.float32(2.0 ** -(8 + 6 * lev))

            @plsc.parallel_loop(0, _NB, unroll=4)
            def _zero(b):
                hcnt[pl.ds(b * _L, _L)] = zeros
                hsum[pl.ds(b * _L, _L)] = zeros

            if lev == 0:
                @plsc.parallel_loop(0, _NCH, unroll=4)
                def _hist0(i):
                    v = buf[pl.ds(i * _L, _L)]
                    b = jnp.minimum((hi - v) * inv_bw, _NB - 1.0).astype(jnp.int32)
                    addr = lanes * _NB + b
                    plsc.addupdate_scatter(hcnt, [addr], ones)
                    plsc.addupdate_scatter(hsum, [addr], v)

                c_top = zeros
                s_top = zeros
            else:
                @plsc.parallel_loop(0, _NCH, carry=(zeros, zeros), unroll=4)
                def _hist(i, carry):
                    ca, sa = carry
                    v = buf[pl.ds(i * _L, _L)]
                    b = jnp.clip((hi - v) * inv_bw, 0.0, _NB - 1.0).astype(jnp.int32)
                    addr = lanes * _NB + b
                    mask = (v <= hi) & (v >= lo)
                    plsc.addupdate_scatter(hcnt, [addr], ones, mask=mask)
                    plsc.addupdate_scatter(hsum, [addr], v, mask=mask)
                    above = v > hi
                    return (ca + jnp.where(above, 1.0, 0.0),
                            sa + jnp.where(above, v, 0.0))

                ca, sa = _hist
                c_top = _splat_last(plsc.cumsum(ca))
                s_top = _splat_last(plsc.cumsum(sa))

            def _scan(g, carry):
                cc0, ss0, bstar, found, ccs, sss = carry
                base = g * _L
                cntv = hcnt[pl.ds(base, _L)]
                smv = hsum[pl.ds(base, _L)]
                for lane in range(1, _L):
                    cntv = cntv + hcnt[pl.ds(lane * _NB + base, _L)]
                    smv = smv + hsum[pl.ds(lane * _NB + base, _L)]
                ccv = cc0 + plsc.cumsum(cntv)
                ssv = ss0 + plsc.cumsum(smv)
                t_edge = hi - ((base + 1).astype(jnp.float32) + lanes_f) * bw
                f = (s_top + ssv) - (c_top + ccv) * t_edge - 1.0
                idx = plsc.all_reduce_ffs(f >= 0.0)
                hit = (idx < _L) & jnp.logical_not(found)
                idxc = jnp.minimum(idx, _L - 1)
                ccv_at = ccv.at[idxc].get(mode="promise_in_bounds")
                ssv_at = ssv.at[idxc].get(mode="promise_in_bounds")
                bstar = jnp.where(hit, base + idx, bstar)
                ccs = jnp.where(hit, c_top + ccv_at, ccs)
                sss = jnp.where(hit, s_top + ssv_at, sss)
                found = found | (idx < _L)
                cc0 = _splat_last(ccv)
                ss0 = _splat_last(ssv)
                return cc0, ss0, bstar, found, ccs, sss

            cc0, ss0, bstar, found, cc_star, ss_star = lax.fori_loop(
                0, _NG, _scan,
                (zeros, zeros, jnp.full((_L,), _NB - 1, jnp.int32),
                 jnp.zeros((_L,), jnp.bool_), ones, zeros))
            cc_star = jnp.where(found, cc_star, c_top + cc0)
            ss_star = jnp.where(found, ss_star, s_top + ss0)
            bsf = bstar.astype(jnp.float32)
            lo = hi - (bsf + 3.0) * bw
            hi = hi - (bsf - 1.0) * bw

        tau = (ss_star - 1.0) / cc_star

        @plsc.parallel_loop(0, _NCH, unroll=4)
        def _out(i):
            v = buf[pl.ds(i * _L, _L)]
            buf[pl.ds(i * _L, _L)] = jnp.maximum(v - tau, 0.0)

        pltpu.sync_copy(buf, out_hbm.at[row])


def _make_sc_kernel():
    mesh = plsc.VectorSubcoreMesh(
        core_axis_name="c", subcore_axis_name="s",
        num_cores=2, num_subcores=16)
    return pl.kernel(
        _sc_body,
        out_type=jax.ShapeDtypeStruct((_R, _N), jnp.float32),
        mesh=mesh,
        scratch_types=[
            pltpu.VMEM((_N,), jnp.float32),
            pltpu.VMEM((_NB * _L,), jnp.float32),
            pltpu.VMEM((_NB * _L,), jnp.float32),
        ],
        compiler_params=pltpu.CompilerParams(needs_layout_passes=False),
    )


@jax.jit
def kernel(input):
    return _make_sc_kernel()(input)


# speedup vs baseline: 2.1190x; 2.1190x over previous
import jax
import jax.numpy as jnp
from jax import lax
from jax.experimental import pallas as pl
from jax.experimental.pallas import tpu as pltpu
from jax.experimental.pallas import tpu_sc as plsc

_R, _N = 128, 32768
_L = 16
_NCH = _N // _L
_NW = 32
_RPW = _R // _NW
_NB = 256
_LEVELS = 2


def _sc_body(in_hbm, out_hbm, buf, hcnt, hsum):
    cid = lax.axis_index("c")
    sid = lax.axis_index("s")
    wid = sid * 2 + cid
    lanes = lax.iota(jnp.int32, _L)
    zeros = jnp.zeros((_L,), jnp.float32)
    ones = jnp.ones((_L,), jnp.float32)

    for r in range(_RPW):
        row = wid * _RPW + r
        pltpu.sync_copy(in_hbm.at[row], buf)

        @plsc.parallel_loop(0, _NCH, carry=jnp.full((_L,), -jnp.inf, jnp.float32), unroll=4)
        def _mx(i, acc):
            return jnp.maximum(acc, buf[pl.ds(i * _L, _L)])

        m = jnp.max(_mx)
        lo = m - 1.0
        hi = m
        cc_star = jnp.float32(1.0)
        ss_star = jnp.float32(0.0)

        for lev in range(_LEVELS):
            inv_bw = jnp.float32(2.0 ** (8 + 6 * lev))
            bw = jnp.float32(2.0 ** -(8 + 6 * lev))

            @plsc.parallel_loop(0, _NB, unroll=4)
            def _zero(b):
                hcnt[pl.ds(b * _L, _L)] = zeros
                hsum[pl.ds(b * _L, _L)] = zeros

            if lev == 0:
                @plsc.parallel_loop(0, _NCH, unroll=4)
                def _hist0(i):
                    v = buf[pl.ds(i * _L, _L)]
                    b = jnp.minimum((hi - v) * inv_bw, _NB - 1.0).astype(jnp.int32)
                    addr = b * _L + lanes
                    plsc.addupdate_scatter(hcnt, [addr], ones)
                    plsc.addupdate_scatter(hsum, [addr], v)

                c_top = jnp.float32(0.0)
                s_top = jnp.float32(0.0)
            else:
                @plsc.parallel_loop(0, _NCH, carry=(zeros, zeros), unroll=4)
                def _hist(i, carry):
                    ca, sa = carry
                    v = buf[pl.ds(i * _L, _L)]
                    b = jnp.clip((hi - v) * inv_bw, 0.0, _NB - 1.0).astype(jnp.int32)
                    addr = b * _L + lanes
                    mask = (v <= hi) & (v >= lo)
                    plsc.addupdate_scatter(hcnt, [addr], ones, mask=mask)
                    plsc.addupdate_scatter(hsum, [addr], v, mask=mask)
                    above = v > hi
                    return (ca + jnp.where(above, 1.0, 0.0),
                            sa + jnp.where(above, v, 0.0))

                ca, sa = _hist
                c_top = jnp.sum(ca)
                s_top = jnp.sum(sa)

            @plsc.parallel_loop(
                0, _NB, unroll=4,
                carry=(jnp.zeros((), jnp.float32), jnp.zeros((), jnp.float32),
                       jnp.full((), _NB - 1, jnp.int32),
                       jnp.zeros((), jnp.bool_),
                       jnp.ones((), jnp.float32), jnp.zeros((), jnp.float32)))
            def _scan(b, carry):
                cc, ss, bstar, found, ccs, sss = carry
                cc = cc + jnp.sum(hcnt[pl.ds(b * _L, _L)])
                ss = ss + jnp.sum(hsum[pl.ds(b * _L, _L)])
                t_edge = hi - (b + 1).astype(jnp.float32) * bw
                f = (s_top + ss) - (c_top + cc) * t_edge - 1.0
                hit = (f >= 0.0) & jnp.logical_not(found)
                bstar = jnp.where(hit, b, bstar)
                ccs = jnp.where(hit, c_top + cc, ccs)
                sss = jnp.where(hit, s_top + ss, sss)
                return cc, ss, bstar, found | hit, ccs, sss

            cc, ss, bstar, found, cc_star, ss_star = _scan
            cc_star = jnp.where(found, cc_star, c_top + cc)
            ss_star = jnp.where(found, ss_star, s_top + ss)
            bsf = bstar.astype(jnp.float32)
            lo = hi - (bsf + 3.0) * bw
            hi = hi - (bsf - 1.0) * bw

        tau = jnp.full((_L,), ss_star - 1.0) / jnp.full((_L,), cc_star)

        @plsc.parallel_loop(0, _NCH, unroll=4)
        def _out(i):
            v = buf[pl.ds(i * _L, _L)]
            buf[pl.ds(i * _L, _L)] = jnp.maximum(v - tau, 0.0)

        pltpu.sync_copy(buf, out_hbm.at[row])


def _make_sc_kernel():
    mesh = plsc.VectorSubcoreMesh(
        core_axis_name="c", subcore_axis_name="s",
        num_cores=2, num_subcores=16)
    return pl.kernel(
        _sc_body,
        out_type=jax.ShapeDtypeStruct((_R, _N), jnp.float32),
        mesh=mesh,
        scratch_types=[
            pltpu.VMEM((_N,), jnp.float32),
            pltpu.VMEM((_NB * _L,), jnp.float32),
            pltpu.VMEM((_NB * _L,), jnp.float32),
        ],
        compiler_params=pltpu.CompilerParams(needs_layout_passes=False),
    )


@jax.jit
def kernel(input):
    return _make_sc_kernel()(input)
